# Initial kernel scaffold; baseline (speedup 1.0000x reference)
import jax
import jax.numpy as jnp
from jax.experimental import pallas as pl


def kernel(x, weight_values, weight_indices):
    raise NotImplementedError("write your pallas kernel here")


# speedup vs baseline: 2.7521x; 2.7521x over previous
import functools

import jax
import jax.numpy as jnp
from jax import lax
from jax.experimental import pallas as pl
from jax.experimental.pallas import tpu as pltpu
from jax.experimental.pallas import tpu_sc as plsc

_B_BLOCKS = 1024
_BLOCK = 16
_BATCH = 64
_ROW_W = _BLOCK * _BATCH
_NC = 2
_NS = 16
_NW = _NC * _NS
_ROWS_PER_W = _B_BLOCKS // _NW
_GROUP = 16


def _sc_body(rhs_hbm, wv_hbm, cols_hbm, lrows_hbm, order_hbm, bounds_hbm,
             out_hbm, xbuf, wbuf, cols_v, lrows_v, order_v, bvec, acc_v,
             gsem, *, nnz_pad):
    cid = lax.axis_index("c")
    sid = lax.axis_index("s")
    wid = sid * _NC + cid

    def _zrow(r, carry):
        def _zcol(q, carry2):
            acc_v[r, pl.ds(q * 16, 16)] = jnp.zeros((16,), jnp.float32)
            return carry2
        return lax.fori_loop(0, _ROW_W // 16, _zcol, carry)
    lax.fori_loop(0, _ROWS_PER_W, _zrow, 0)

    pltpu.sync_copy(cols_hbm, cols_v)
    pltpu.sync_copy(lrows_hbm, lrows_v)
    pltpu.sync_copy(order_hbm, order_v)
    pltpu.sync_copy(bounds_hbm.at[wid], bvec)
    bv = bvec[...]
    start = bv[0]
    end = bv[1]
    n_groups = (end - start + (_GROUP - 1)) // _GROUP

    def _group(g, carry):
        pos = start + g * _GROUP
        cvec = cols_v[pl.ds(pos, _GROUP)]
        ovec = order_v[pl.ds(pos, _GROUP)]
        cp_x = pltpu.async_copy(rhs_hbm.at[cvec], xbuf, gsem)
        cp_w = pltpu.async_copy(wv_hbm.at[ovec], wbuf, gsem)
        cp_x.wait()
        cp_w.wait()

        def _block(b, carry2):
            valid = pos + b < end
            lr = lrows_v[pl.ds(pos + b, 16)][0]

            @pl.when(valid)
            def _():
                for kh in range(2):
                    xv = [[xbuf[b, pl.ds((kh * 8 + k) * _BATCH + q * 16, 16)]
                           for q in range(4)] for k in range(8)]
                    for i in range(_BLOCK):
                        wrow = wbuf[b, pl.ds(i * _BLOCK, _BLOCK)]
                        acc = [acc_v[lr, pl.ds(i * _BATCH + q * 16, 16)]
                               for q in range(4)]
                        for k in range(8):
                            w = wrow[kh * 8 + k]
                            for q in range(4):
                                acc[q] = acc[q] + w * xv[k][q]
                        for q in range(4):
                            acc_v[lr, pl.ds(i * _BATCH + q * 16, 16)] = acc[q]
            return carry2
        lax.fori_loop(0, _GROUP, _block, 0)
        return carry
    lax.fori_loop(0, n_groups, _group, 0)

    pltpu.sync_copy(acc_v, out_hbm.at[pl.ds(wid * _ROWS_PER_W, _ROWS_PER_W)])


def _merge_body(p_ref, o_ref):
    o_ref[...] = p_ref[...].T


def kernel(x, weight_values, weight_indices):
    nnz = weight_values.shape[0]
    nnz_pad = -(-nnz // _GROUP) * _GROUP

    rhs = x.T.reshape(_B_BLOCKS, _ROW_W)
    wv = weight_values.reshape(nnz, _BLOCK * _BLOCK)

    rows = weight_indices[0]
    cols = weight_indices[1]
    order = jnp.argsort(rows)
    srows = rows[order]
    scols = cols[order]
    pad = nnz_pad - nnz
    order_p = jnp.pad(order, (0, pad))
    scols_p = jnp.pad(scols, (0, pad))
    lrows_p = jnp.pad(srows - (srows // _ROWS_PER_W) * _ROWS_PER_W,
                      (0, pad + _GROUP))
    edges = jnp.searchsorted(srows, jnp.arange(_NW + 1, dtype=jnp.int32) * _ROWS_PER_W)
    bounds = jnp.stack([edges[:-1], edges[1:]], axis=1).astype(jnp.int32)
    bounds = jnp.pad(bounds, ((0, 0), (0, 14)))

    mesh = plsc.VectorSubcoreMesh(core_axis_name="c", subcore_axis_name="s")
    partial = pl.kernel(
        functools.partial(_sc_body, nnz_pad=nnz_pad),
        out_type=jax.ShapeDtypeStruct((_B_BLOCKS, _ROW_W), jnp.float32),
        mesh=mesh,
        scratch_types=[
            pltpu.VMEM((_GROUP, _ROW_W), jnp.float32),
            pltpu.VMEM((_GROUP, _BLOCK * _BLOCK), jnp.float32),
            pltpu.VMEM((nnz_pad,), jnp.int32),
            pltpu.VMEM((nnz_pad + _GROUP,), jnp.int32),
            pltpu.VMEM((nnz_pad,), jnp.int32),
            pltpu.VMEM((16,), jnp.int32),
            pltpu.VMEM((_ROWS_PER_W, _ROW_W), jnp.float32),
            pltpu.SemaphoreType.DMA,
        ],
    )(rhs, wv, scols_p, lrows_p, order_p, bounds)

    p = partial.reshape(_B_BLOCKS * _BLOCK, _BATCH)
    out = pl.pallas_call(
        _merge_body,
        grid=(32,),
        in_specs=[pl.BlockSpec((512, _BATCH), lambda i: (i, 0))],
        out_specs=pl.BlockSpec((_BATCH, 512), lambda i: (0, i)),
        out_shape=jax.ShapeDtypeStruct((_BATCH, _B_BLOCKS * _BLOCK), jnp.float32),
    )(p)
    return out


# speedup vs baseline: 3.9429x; 1.4327x over previous
import functools

import jax
import jax.numpy as jnp
from jax import lax
from jax.experimental import pallas as pl
from jax.experimental.pallas import tpu as pltpu
from jax.experimental.pallas import tpu_sc as plsc

_B_BLOCKS = 1024
_BLOCK = 16
_BATCH = 64
_ROW_W = _BLOCK * _BATCH
_NC = 2
_NS = 16
_NW = _NC * _NS
_ROWS_PER_W = _B_BLOCKS // _NW
_GROUP = 16


def _sc_body(rhs_hbm, wv_hbm, cols_hbm, lrows_hbm, order_hbm, bounds_hbm,
             out_hbm, xbuf, wbuf, cols_v, lrows_v, order_v, bvec, acc_v,
             xs0, ws0, xs1, ws1, *, nnz_pad):
    cid = lax.axis_index("c")
    sid = lax.axis_index("s")
    wid = sid * _NC + cid

    def _zrow(r, carry):
        def _zcol(q, carry2):
            acc_v[r, pl.ds(q * 16, 16)] = jnp.zeros((16,), jnp.float32)
            return carry2
        return lax.fori_loop(0, _ROW_W // 16, _zcol, carry)
    lax.fori_loop(0, _ROWS_PER_W, _zrow, 0)

    pltpu.sync_copy(cols_hbm, cols_v)
    pltpu.sync_copy(lrows_hbm, lrows_v)
    pltpu.sync_copy(order_hbm, order_v)
    pltpu.sync_copy(bounds_hbm.at[wid], bvec)
    bv = bvec[...]
    start = bv[0]
    end = bv[1]
    n_groups = (end - start + (_GROUP - 1)) // _GROUP
    gmax = jnp.maximum(n_groups - 1, 0)

    def _gpos(g):
        return start + jnp.minimum(g, gmax) * _GROUP

    def _issue(g, slot, xs, ws):
        pos = _gpos(g)
        pltpu.async_copy(rhs_hbm.at[cols_v[pl.ds(pos, _GROUP)]],
                         xbuf.at[slot], xs)
        pltpu.async_copy(wv_hbm.at[order_v[pl.ds(pos, _GROUP)]],
                         wbuf.at[slot], ws)

    def _wait(g, slot, xs, ws):
        pos = _gpos(g)
        pltpu.make_async_copy(rhs_hbm.at[cols_v[pl.ds(pos, _GROUP)]],
                              xbuf.at[slot], xs).wait()
        pltpu.make_async_copy(wv_hbm.at[order_v[pl.ds(pos, _GROUP)]],
                              wbuf.at[slot], ws).wait()

    def _compute(g, slot):
        pos = _gpos(g)
        real = start + g * _GROUP

        def _block(b, carry2):
            valid = real + b < end
            lr = lrows_v[pl.ds(pos + b, 16)][0]

            @pl.when(valid)
            def _():
                for kh in range(2):
                    xv = [[xbuf[slot, b,
                                pl.ds((kh * 8 + k) * _BATCH + q * 16, 16)]
                           for q in range(4)] for k in range(8)]
                    for i in range(_BLOCK):
                        wrow = wbuf[slot, b, pl.ds(i * _BLOCK, _BLOCK)]
                        acc = [acc_v[lr, pl.ds(i * _BATCH + q * 16, 16)]
                               for q in range(4)]
                        for k in range(8):
                            w = wrow[kh * 8 + k]
                            for q in range(4):
                                acc[q] = acc[q] + w * xv[k][q]
                        for q in range(4):
                            acc_v[lr, pl.ds(i * _BATCH + q * 16, 16)] = acc[q]
            return carry2
        lax.fori_loop(0, _GROUP, _block, 0)

    @pl.when(n_groups > 0)
    def _():
        _issue(0, 0, xs0, ws0)

    def _pair(h, carry):
        g0 = 2 * h
        g1 = g0 + 1
        _issue(g1, 1, xs1, ws1)
        _wait(g0, 0, xs0, ws0)
        _compute(g0, 0)
        _issue(g0 + 2, 0, xs0, ws0)
        _wait(g1, 1, xs1, ws1)
        _compute(g1, 1)
        return carry
    lax.fori_loop(0, (n_groups + 1) // 2, _pair, 0)

    @pl.when(n_groups > 0)
    def _():
        _wait(0, 0, xs0, ws0)

    pltpu.sync_copy(acc_v, out_hbm.at[pl.ds(wid * _ROWS_PER_W, _ROWS_PER_W)])


def _merge_body(p_ref, o_ref):
    o_ref[...] = p_ref[...].T


def kernel(x, weight_values, weight_indices):
    nnz = weight_values.shape[0]
    nnz_pad = -(-nnz // _GROUP) * _GROUP

    rhs = x.T.reshape(_B_BLOCKS, _ROW_W)
    wv = weight_values.reshape(nnz, _BLOCK * _BLOCK)

    rows = weight_indices[0]
    cols = weight_indices[1]
    srows, scols, order = lax.sort(
        (rows, cols, jnp.arange(nnz, dtype=jnp.int32)), num_keys=1)
    pad = nnz_pad - nnz
    order_p = jnp.pad(order, (0, pad))
    scols_p = jnp.pad(scols, (0, pad))
    lrows_p = jnp.pad(srows - (srows // _ROWS_PER_W) * _ROWS_PER_W,
                      (0, pad + _GROUP))
    edges = jnp.searchsorted(srows, jnp.arange(_NW + 1, dtype=jnp.int32) * _ROWS_PER_W)
    bounds = jnp.stack([edges[:-1], edges[1:]], axis=1).astype(jnp.int32)
    bounds = jnp.pad(bounds, ((0, 0), (0, 14)))

    mesh = plsc.VectorSubcoreMesh(core_axis_name="c", subcore_axis_name="s")
    partial = pl.kernel(
        functools.partial(_sc_body, nnz_pad=nnz_pad),
        out_type=jax.ShapeDtypeStruct((_B_BLOCKS, _ROW_W), jnp.float32),
        mesh=mesh,
        scratch_types=[
            pltpu.VMEM((2, _GROUP, _ROW_W), jnp.float32),
            pltpu.VMEM((2, _GROUP, _BLOCK * _BLOCK), jnp.float32),
            pltpu.VMEM((nnz_pad,), jnp.int32),
            pltpu.VMEM((nnz_pad + _GROUP,), jnp.int32),
            pltpu.VMEM((nnz_pad,), jnp.int32),
            pltpu.VMEM((16,), jnp.int32),
            pltpu.VMEM((_ROWS_PER_W, _ROW_W), jnp.float32),
            pltpu.SemaphoreType.DMA,
            pltpu.SemaphoreType.DMA,
            pltpu.SemaphoreType.DMA,
            pltpu.SemaphoreType.DMA,
        ],
    )(rhs, wv, scols_p, lrows_p, order_p, bounds)

    p = partial.reshape(_B_BLOCKS * _BLOCK, _BATCH)
    out = pl.pallas_call(
        _merge_body,
        grid=(8,),
        in_specs=[pl.BlockSpec((2048, _BATCH), lambda i: (i, 0))],
        out_specs=pl.BlockSpec((_BATCH, 2048), lambda i: (0, i)),
        out_shape=jax.ShapeDtypeStruct((_BATCH, _B_BLOCKS * _BLOCK), jnp.float32),
    )(p)
    return out
